# baseline (speedup 1.0000x reference)
import jax
import jax.numpy as jnp
from jax import lax
from jax.experimental import pallas as pl
from jax.experimental.pallas import tpu as pltpu
from jax.experimental.pallas import tpu_sc as plsc

N = 10000
E = 160000
IN_DIM = 256
HEADS = 8
DPH = 32
QKV = HEADS * DPH

NC = 2
NS = 16
L = 16

HPC = HEADS // NC
CW = HPC * DPH

EPT = E // NS
C = 48
BF = 16
NCHUNK = EPT // C
TAILE = EPT - NCHUNK * C
NPAIR = NCHUNK // 2

RPT = N // NS
NRPT = 624
NRCH = NRPT // C
NTAIL = N - NS * NRPT

_INV_SQRT_D = float(DPH) ** -0.5
UNR = 8


_BN = 1000


def _mm_body(x_ref, w_ref, b_ref, q_ref, k_ref, v_ref):
    acc = jnp.dot(x_ref[...], w_ref[...], preferred_element_type=jnp.float32)
    acc = acc + b_ref[...]
    for half, ref in enumerate((q_ref, k_ref, v_ref)):
        ref[0] = acc[:, 2 * half * CW:(2 * half + 1) * CW]
        ref[1] = acc[:, (2 * half + 1) * CW:(2 * half + 2) * CW]


def _tc_qkv(x, wc, bc):
    return pl.pallas_call(
        _mm_body,
        grid=(N // _BN,),
        in_specs=[
            pl.BlockSpec((_BN, IN_DIM), lambda i: (i, 0)),
            pl.BlockSpec((IN_DIM, 3 * QKV), lambda i: (0, 0)),
            pl.BlockSpec((1, 3 * QKV), lambda i: (0, 0)),
        ],
        out_specs=[pl.BlockSpec((NC, _BN, CW), lambda i: (0, i, 0))] * 3,
        out_shape=[jax.ShapeDtypeStruct((NC, N, CW), jnp.float32)] * 3,
    )(x, wc, bc)


def _sc_body(q_hbm, k_hbm, v_hbm, src_hbm, dst_hbm, z1_hbm, z2_hbm, out_hbm,
             qb0, kb0, vb0, qb1, kb1, vb1, den0, den1,
             db0, db1, sb0, sb1, dbt, sbt, sbig, dbig,
             acc, dacc, gsem0, gsem1, ssem0, ssem1):
    c = lax.axis_index("c")
    s = lax.axis_index("s")
    qp = q_hbm.at[c]
    kp = k_hbm.at[c]
    vp = v_hbm.at[c]
    qb = (qb0, qb1)
    kb = (kb0, kb1)
    vb = (vb0, vb1)
    den = (den0, den1)
    db = (db0, db1)
    sb = (sb0, sb1)
    gsem = (gsem0, gsem1)
    ssem = (ssem0, ssem1)

    pltpu.sync_copy(z1_hbm, acc.at[pl.ds(s * RPT, RPT)])
    pltpu.sync_copy(z2_hbm, dacc.at[pl.ds(s * RPT, RPT)])
    pltpu.sync_copy(z2_hbm.at[pl.ds(0, C)], den0)
    pltpu.sync_copy(z2_hbm.at[pl.ds(0, C)], den1)
    plsc.subcore_barrier()

    iot = lax.iota(jnp.int32, L)
    ebase = s * EPT

    def load_idx(ci, b):
        k = ci & (BF - 1)

        @pl.when(k == 0)
        def _():
            off = pl.multiple_of(ebase + ci * C, 8)
            pltpu.sync_copy(src_hbm.at[pl.ds(off, BF * C)], sbig)
            pltpu.sync_copy(dst_hbm.at[pl.ds(off, BF * C)], dbig)

        for m in range(C // L):
            sl = pl.ds(k * C + m * L, L)
            sb[b][0, pl.ds(m * L, L)] = sbig[sl]
            db[b][0, pl.ds(m * L, L)] = dbig[sl]

    def fire_gathers(b):
        pltpu.async_copy(qp.at[db[b].at[0]], qb[b], gsem[b])
        pltpu.async_copy(kp.at[sb[b].at[0]], kb[b], gsem[b])
        pltpu.async_copy(vp.at[sb[b].at[0]], vb[b], gsem[b])

    def drain_gathers(b):
        pltpu.make_async_copy(qp.at[db[b].at[0]], qb[b], gsem[b]).wait()
        pltpu.make_async_copy(kp.at[sb[b].at[0]], kb[b], gsem[b]).wait()
        pltpu.make_async_copy(vp.at[sb[b].at[0]], vb[b], gsem[b]).wait()

    def fire_scatters(b):
        pltpu.async_copy(vb[b], acc.at[db[b].at[0]], ssem[b], add=True)
        pltpu.async_copy(den[b], dacc.at[db[b].at[0]], ssem[b], add=True)

    def drain_scatters(b):
        pltpu.make_async_copy(vb[b], acc.at[pl.ds(0, C)], ssem[b]).wait()
        pltpu.make_async_copy(den[b], dacc.at[pl.ds(0, C)], ssem[b]).wait()

    def compute(b, ngrp):
        def grp(g, carry2):
            er = g * L + iot

            def score_m(mq, saccs):
                out = list(saccs)
                for u in range(UNR):
                    colp = iot ^ (mq * UNR + u)
                    for h in range(HPC):
                        sacc = out[h]
                        for hi in (0, L):
                            col = colp + (h * DPH + hi)
                            qv = plsc.load_gather(qb[b], [er, col])
                            kv = plsc.load_gather(kb[b], [er, col])
                            sacc = sacc + qv * kv
                        out[h] = sacc
                return tuple(out)

            zero = jnp.zeros((L,), jnp.float32)
            saccs = lax.fori_loop(0, L // UNR, score_m, (zero,) * HPC)
            ws = []
            for h in range(HPC):
                w = jnp.exp(jnp.clip(saccs[h] * _INV_SQRT_D, -5.0, 5.0))
                ws.append(w)
                plsc.store_scatter(den[b], [er, jnp.full((L,), h, jnp.int32)], w)

            def msg_m(mq, carry3):
                for u in range(UNR):
                    colp = iot ^ (mq * UNR + u)
                    for h in range(HPC):
                        for hi in (0, L):
                            col = colp + (h * DPH + hi)
                            vv = plsc.load_gather(vb[b], [er, col])
                            plsc.store_scatter(vb[b], [er, col], vv * ws[h])
                return carry3

            lax.fori_loop(0, L // UNR, msg_m, 0)
            return carry2

        lax.fori_loop(0, ngrp, grp, 0)

    load_idx(0, 0)
    fire_gathers(0)

    def pair(it, carry):
        @pl.when(it > 0)
        def _():
            drain_scatters(1)
        load_idx(2 * it + 1, 1)
        fire_gathers(1)
        drain_gathers(0)
        compute(0, C // L)
        fire_scatters(0)
        @pl.when(it < NPAIR - 1)
        def _():
            drain_scatters(0)
            load_idx(2 * it + 2, 0)
            fire_gathers(0)
        drain_gathers(1)
        compute(1, C // L)
        fire_scatters(1)
        return carry

    lax.fori_loop(0, NPAIR, pair, 0)

    drain_scatters(0)
    toff = pl.multiple_of(ebase + NCHUNK * C, 8)
    pltpu.sync_copy(src_hbm.at[pl.ds(toff, TAILE)], sbt.at[0])
    pltpu.sync_copy(dst_hbm.at[pl.ds(toff, TAILE)], dbt.at[0])
    cp1 = pltpu.async_copy(qp.at[dbt.at[0]], qb0.at[pl.ds(0, TAILE)], gsem0)
    cp2 = pltpu.async_copy(kp.at[sbt.at[0]], kb0.at[pl.ds(0, TAILE)], gsem0)
    cp3 = pltpu.async_copy(vp.at[sbt.at[0]], vb0.at[pl.ds(0, TAILE)], gsem0)
    cp1.wait()
    cp2.wait()
    cp3.wait()
    compute(0, TAILE // L)
    pltpu.async_copy(vb0.at[pl.ds(0, TAILE)], acc.at[dbt.at[0]], ssem0, add=True)
    pltpu.async_copy(den0.at[pl.ds(0, TAILE)], dacc.at[dbt.at[0]], ssem0, add=True)
    pltpu.make_async_copy(vb0.at[pl.ds(0, TAILE)], acc.at[pl.ds(0, TAILE)], ssem0).wait()
    pltpu.make_async_copy(den0.at[pl.ds(0, TAILE)], dacc.at[pl.ds(0, TAILE)], ssem0).wait()
    drain_scatters(1)
    plsc.subcore_barrier()

    def norm_rows(rn, nrows):
        pltpu.sync_copy(acc.at[pl.ds(rn, nrows)], qb0.at[pl.ds(0, nrows)])
        pltpu.sync_copy(dacc.at[pl.ds(rn, nrows)], den0.at[pl.ds(0, nrows)])
        for g2 in range(nrows // L):
            er2 = g2 * L + iot
            recs = [1.0 / (plsc.load_gather(
                den0, [er2, jnp.full((L,), h, jnp.int32)]) + 1e-16)
                    for h in range(HPC)]

            def nm(m, carry2):
                colp = iot ^ m
                for jv in range(CW // L):
                    col = colp + jv * L
                    v = plsc.load_gather(qb0, [er2, col])
                    plsc.store_scatter(qb0, [er2, col], v * recs[jv // 2])
                return carry2

            lax.fori_loop(0, L, nm, 0)
        pltpu.sync_copy(qb0.at[pl.ds(0, nrows)],
                        out_hbm.at[pl.ds(rn, nrows), pl.ds(c * CW, CW)])

    def norm(ch, carry):
        norm_rows(s * NRPT + ch * C, C)
        return carry

    lax.fori_loop(0, NRCH, norm, 0)

    @pl.when(s == NS - 1)
    def _norm_tail():
        norm_rows(NS * NRPT, NTAIL)


_sc_attn = pl.kernel(
    _sc_body,
    out_type=jax.ShapeDtypeStruct((N, QKV), jnp.float32),
    mesh=plsc.VectorSubcoreMesh(core_axis_name="c", subcore_axis_name="s",
                                num_cores=NC, num_subcores=NS),
    scratch_types=[
        pltpu.VMEM((C, CW), jnp.float32),
        pltpu.VMEM((C, CW), jnp.float32),
        pltpu.VMEM((C, CW), jnp.float32),
        pltpu.VMEM((C, CW), jnp.float32),
        pltpu.VMEM((C, CW), jnp.float32),
        pltpu.VMEM((C, CW), jnp.float32),
        pltpu.VMEM((C, 8), jnp.float32),
        pltpu.VMEM((C, 8), jnp.float32),
        pltpu.VMEM((1, C), jnp.int32),
        pltpu.VMEM((1, C), jnp.int32),
        pltpu.VMEM((1, C), jnp.int32),
        pltpu.VMEM((1, C), jnp.int32),
        pltpu.VMEM((1, 16), jnp.int32),
        pltpu.VMEM((1, 16), jnp.int32),
        pltpu.VMEM((BF * C,), jnp.int32),
        pltpu.VMEM((BF * C,), jnp.int32),
        pltpu.VMEM_SHARED((N, CW), jnp.float32),
        pltpu.VMEM_SHARED((N, 8), jnp.float32),
        pltpu.SemaphoreType.DMA,
        pltpu.SemaphoreType.DMA,
        pltpu.SemaphoreType.DMA,
        pltpu.SemaphoreType.DMA,
    ],
    compiler_params=pltpu.CompilerParams(use_tc_tiling_on_sc=False,
                                         needs_layout_passes=False),
)


def kernel(x, edge_index, Wq, bq, Wk, bk, Wv, bv):
    wc = jnp.concatenate([Wq, Wk, Wv], axis=1)
    bc = jnp.concatenate([bq, bk, bv]).reshape(1, 3 * QKV)
    q, k, v = _tc_qkv(x, wc, bc)
    z1 = jnp.zeros((RPT, CW), jnp.float32)
    z2 = jnp.zeros((RPT, 8), jnp.float32)
    return _sc_attn(q, k, v, edge_index[0], edge_index[1], z1, z2)


# speedup vs baseline: 1.2873x; 1.2873x over previous
import jax
import jax.numpy as jnp
from jax import lax
from jax.experimental import pallas as pl
from jax.experimental.pallas import tpu as pltpu
from jax.experimental.pallas import tpu_sc as plsc

N = 10000
E = 160000
IN_DIM = 256
HEADS = 8
DPH = 32
QKV = HEADS * DPH

NC = 2
NS = 16
L = 16

HPC = HEADS // NC
CW = HPC * DPH

EPT = E // NS
C = 48
BF = 16
NCHUNK = EPT // C
TAILE = EPT - NCHUNK * C
NPAIR = NCHUNK // 2

RPT = N // NS
NRPT = 624
NRCH = NRPT // C
NTAIL = N - NS * NRPT

_INV_SQRT_D = float(DPH) ** -0.5
UNR = 4


_BN = 1000


def _mm_body(x_ref, w_ref, b_ref, q_ref, k_ref, v_ref):
    acc = jnp.dot(x_ref[...], w_ref[...], preferred_element_type=jnp.float32)
    acc = acc + b_ref[...]
    for half, ref in enumerate((q_ref, k_ref, v_ref)):
        ref[0] = acc[:, 2 * half * CW:(2 * half + 1) * CW]
        ref[1] = acc[:, (2 * half + 1) * CW:(2 * half + 2) * CW]


def _tc_qkv(x, wc, bc):
    return pl.pallas_call(
        _mm_body,
        grid=(N // _BN,),
        in_specs=[
            pl.BlockSpec((_BN, IN_DIM), lambda i: (i, 0)),
            pl.BlockSpec((IN_DIM, 3 * QKV), lambda i: (0, 0)),
            pl.BlockSpec((1, 3 * QKV), lambda i: (0, 0)),
        ],
        out_specs=[pl.BlockSpec((NC, _BN, CW), lambda i: (0, i, 0))] * 3,
        out_shape=[jax.ShapeDtypeStruct((NC, N, CW), jnp.float32)] * 3,
    )(x, wc, bc)


def _sc_body(q_hbm, k_hbm, v_hbm, src_hbm, dst_hbm, z1_hbm, z2_hbm, out_hbm,
             qb0, kb0, vb0, qb1, kb1, vb1, den0, den1,
             db0, db1, sb0, sb1, dbt, sbt, sbig, dbig,
             acc, dacc, gsem0, gsem1, ssem0, ssem1):
    c = lax.axis_index("c")
    s = lax.axis_index("s")
    qp = q_hbm.at[c]
    kp = k_hbm.at[c]
    vp = v_hbm.at[c]
    qb = (qb0, qb1)
    kb = (kb0, kb1)
    vb = (vb0, vb1)
    den = (den0, den1)
    db = (db0, db1)
    sb = (sb0, sb1)
    gsem = (gsem0, gsem1)
    ssem = (ssem0, ssem1)

    pltpu.sync_copy(z1_hbm, acc.at[pl.ds(s * RPT, RPT)])
    pltpu.sync_copy(z2_hbm, dacc.at[pl.ds(s * RPT, RPT)])
    pltpu.sync_copy(z2_hbm.at[pl.ds(0, C)], den0)
    pltpu.sync_copy(z2_hbm.at[pl.ds(0, C)], den1)
    plsc.subcore_barrier()

    iot = lax.iota(jnp.int32, L)
    ebase = s * EPT

    def load_idx(ci, b):
        k = ci & (BF - 1)

        @pl.when(k == 0)
        def _():
            off = pl.multiple_of(ebase + ci * C, 8)
            pltpu.sync_copy(src_hbm.at[pl.ds(off, BF * C)], sbig)
            pltpu.sync_copy(dst_hbm.at[pl.ds(off, BF * C)], dbig)

        for m in range(C // L):
            sl = pl.ds(k * C + m * L, L)
            sb[b][0, pl.ds(m * L, L)] = sbig[sl]
            db[b][0, pl.ds(m * L, L)] = dbig[sl]

    def fire_gathers(b):
        pltpu.async_copy(qp.at[db[b].at[0]], qb[b], gsem[b])
        pltpu.async_copy(kp.at[sb[b].at[0]], kb[b], gsem[b])
        pltpu.async_copy(vp.at[sb[b].at[0]], vb[b], gsem[b])

    def drain_gathers(b):
        pltpu.make_async_copy(qp.at[db[b].at[0]], qb[b], gsem[b]).wait()
        pltpu.make_async_copy(kp.at[sb[b].at[0]], kb[b], gsem[b]).wait()
        pltpu.make_async_copy(vp.at[sb[b].at[0]], vb[b], gsem[b]).wait()

    def fire_scatters(b):
        pltpu.async_copy(vb[b], acc.at[db[b].at[0]], ssem[b], add=True)
        pltpu.async_copy(den[b], dacc.at[db[b].at[0]], ssem[b], add=True)

    def drain_scatters(b):
        pltpu.make_async_copy(vb[b], acc.at[pl.ds(0, C)], ssem[b]).wait()
        pltpu.make_async_copy(den[b], dacc.at[pl.ds(0, C)], ssem[b]).wait()

    def compute(b, ngrp):
        def grp(g, carry2):
            er = g * L + iot

            def score_m(mq, saccs):
                out = list(saccs)
                for u in range(UNR):
                    colp = iot ^ (mq * UNR + u)
                    for h in range(HPC):
                        sacc = out[h]
                        for hi in (0, L):
                            col = colp + (h * DPH + hi)
                            qv = plsc.load_gather(qb[b], [er, col])
                            kv = plsc.load_gather(kb[b], [er, col])
                            sacc = sacc + qv * kv
                        out[h] = sacc
                return tuple(out)

            zero = jnp.zeros((L,), jnp.float32)
            saccs = plsc.parallel_loop(0, L // UNR, carry=(zero,) * HPC)(score_m)
            ws = []
            for h in range(HPC):
                w = jnp.exp(jnp.clip(saccs[h] * _INV_SQRT_D, -5.0, 5.0))
                ws.append(w)
                plsc.store_scatter(den[b], [er, jnp.full((L,), h, jnp.int32)], w)

            def msg_m(mq, carry3):
                for u in range(UNR):
                    colp = iot ^ (mq * UNR + u)
                    for h in range(HPC):
                        for hi in (0, L):
                            col = colp + (h * DPH + hi)
                            vv = plsc.load_gather(vb[b], [er, col])
                            plsc.store_scatter(vb[b], [er, col], vv * ws[h])
                return carry3

            plsc.parallel_loop(0, L // UNR, carry=jnp.int32(0))(msg_m)
            return carry2

        lax.fori_loop(0, ngrp, grp, 0)

    load_idx(0, 0)
    fire_gathers(0)

    def pair(it, carry):
        @pl.when(it > 0)
        def _():
            drain_scatters(1)
        load_idx(2 * it + 1, 1)
        fire_gathers(1)
        drain_gathers(0)
        compute(0, C // L)
        fire_scatters(0)
        @pl.when(it < NPAIR - 1)
        def _():
            drain_scatters(0)
            load_idx(2 * it + 2, 0)
            fire_gathers(0)
        drain_gathers(1)
        compute(1, C // L)
        fire_scatters(1)
        return carry

    lax.fori_loop(0, NPAIR, pair, 0)

    drain_scatters(0)
    toff = pl.multiple_of(ebase + NCHUNK * C, 8)
    pltpu.sync_copy(src_hbm.at[pl.ds(toff, TAILE)], sbt.at[0])
    pltpu.sync_copy(dst_hbm.at[pl.ds(toff, TAILE)], dbt.at[0])
    cp1 = pltpu.async_copy(qp.at[dbt.at[0]], qb0.at[pl.ds(0, TAILE)], gsem0)
    cp2 = pltpu.async_copy(kp.at[sbt.at[0]], kb0.at[pl.ds(0, TAILE)], gsem0)
    cp3 = pltpu.async_copy(vp.at[sbt.at[0]], vb0.at[pl.ds(0, TAILE)], gsem0)
    cp1.wait()
    cp2.wait()
    cp3.wait()
    compute(0, TAILE // L)
    pltpu.async_copy(vb0.at[pl.ds(0, TAILE)], acc.at[dbt.at[0]], ssem0, add=True)
    pltpu.async_copy(den0.at[pl.ds(0, TAILE)], dacc.at[dbt.at[0]], ssem0, add=True)
    pltpu.make_async_copy(vb0.at[pl.ds(0, TAILE)], acc.at[pl.ds(0, TAILE)], ssem0).wait()
    pltpu.make_async_copy(den0.at[pl.ds(0, TAILE)], dacc.at[pl.ds(0, TAILE)], ssem0).wait()
    drain_scatters(1)
    plsc.subcore_barrier()

    def norm_rows(rn, nrows):
        pltpu.sync_copy(acc.at[pl.ds(rn, nrows)], qb0.at[pl.ds(0, nrows)])
        pltpu.sync_copy(dacc.at[pl.ds(rn, nrows)], den0.at[pl.ds(0, nrows)])
        for g2 in range(nrows // L):
            er2 = g2 * L + iot
            recs = [1.0 / (plsc.load_gather(
                den0, [er2, jnp.full((L,), h, jnp.int32)]) + 1e-16)
                    for h in range(HPC)]

            def nm(m, carry2):
                colp = iot ^ m
                for jv in range(CW // L):
                    col = colp + jv * L
                    v = plsc.load_gather(qb0, [er2, col])
                    plsc.store_scatter(qb0, [er2, col], v * recs[jv // 2])
                return carry2

            lax.fori_loop(0, L, nm, 0)
        pltpu.sync_copy(qb0.at[pl.ds(0, nrows)],
                        out_hbm.at[pl.ds(rn, nrows), pl.ds(c * CW, CW)])

    def norm(ch, carry):
        norm_rows(s * NRPT + ch * C, C)
        return carry

    lax.fori_loop(0, NRCH, norm, 0)

    @pl.when(s == NS - 1)
    def _norm_tail():
        norm_rows(NS * NRPT, NTAIL)


_sc_attn = pl.kernel(
    _sc_body,
    out_type=jax.ShapeDtypeStruct((N, QKV), jnp.float32),
    mesh=plsc.VectorSubcoreMesh(core_axis_name="c", subcore_axis_name="s",
                                num_cores=NC, num_subcores=NS),
    scratch_types=[
        pltpu.VMEM((C, CW), jnp.float32),
        pltpu.VMEM((C, CW), jnp.float32),
        pltpu.VMEM((C, CW), jnp.float32),
        pltpu.VMEM((C, CW), jnp.float32),
        pltpu.VMEM((C, CW), jnp.float32),
        pltpu.VMEM((C, CW), jnp.float32),
        pltpu.VMEM((C, 8), jnp.float32),
        pltpu.VMEM((C, 8), jnp.float32),
        pltpu.VMEM((1, C), jnp.int32),
        pltpu.VMEM((1, C), jnp.int32),
        pltpu.VMEM((1, C), jnp.int32),
        pltpu.VMEM((1, C), jnp.int32),
        pltpu.VMEM((1, 16), jnp.int32),
        pltpu.VMEM((1, 16), jnp.int32),
        pltpu.VMEM((BF * C,), jnp.int32),
        pltpu.VMEM((BF * C,), jnp.int32),
        pltpu.VMEM_SHARED((N, CW), jnp.float32),
        pltpu.VMEM_SHARED((N, 8), jnp.float32),
        pltpu.SemaphoreType.DMA,
        pltpu.SemaphoreType.DMA,
        pltpu.SemaphoreType.DMA,
        pltpu.SemaphoreType.DMA,
    ],
    compiler_params=pltpu.CompilerParams(use_tc_tiling_on_sc=False,
                                         needs_layout_passes=False),
)


def kernel(x, edge_index, Wq, bq, Wk, bk, Wv, bv):
    wc = jnp.concatenate([Wq, Wk, Wv], axis=1)
    bc = jnp.concatenate([bq, bk, bv]).reshape(1, 3 * QKV)
    q, k, v = _tc_qkv(x, wc, bc)
    z1 = jnp.zeros((RPT, CW), jnp.float32)
    z2 = jnp.zeros((RPT, 8), jnp.float32)
    return _sc_attn(q, k, v, edge_index[0], edge_index[1], z1, z2)


# speedup vs baseline: 1.3426x; 1.0430x over previous
import jax
import jax.numpy as jnp
from jax import lax
from jax.experimental import pallas as pl
from jax.experimental.pallas import tpu as pltpu
from jax.experimental.pallas import tpu_sc as plsc

N = 10000
E = 160000
IN_DIM = 256
HEADS = 8
DPH = 32
QKV = HEADS * DPH

NC = 2
NS = 16
L = 16

HPC = HEADS // NC
CW = HPC * DPH

EPT = E // NS
C = 48
BF = 16
NCHUNK = EPT // C
TAILE = EPT - NCHUNK * C
NPAIR = NCHUNK // 2

RPT = N // NS
NRPT = 624
NRCH = NRPT // C
NTAIL = N - NS * NRPT

_INV_SQRT_D = float(DPH) ** -0.5
UNR = 4


_BN = 1000


def _mm_body(x_ref, w_ref, b_ref, q_ref, k_ref, v_ref):
    acc = jnp.dot(x_ref[...], w_ref[...], preferred_element_type=jnp.float32)
    acc = acc + b_ref[...]
    for half, ref in enumerate((q_ref, k_ref, v_ref)):
        ref[0] = acc[:, 2 * half * CW:(2 * half + 1) * CW]
        ref[1] = acc[:, (2 * half + 1) * CW:(2 * half + 2) * CW]


def _tc_qkv(x, wc, bc):
    return pl.pallas_call(
        _mm_body,
        grid=(N // _BN,),
        in_specs=[
            pl.BlockSpec((_BN, IN_DIM), lambda i: (i, 0)),
            pl.BlockSpec((IN_DIM, 3 * QKV), lambda i: (0, 0)),
            pl.BlockSpec((1, 3 * QKV), lambda i: (0, 0)),
        ],
        out_specs=[pl.BlockSpec((NC, _BN, CW), lambda i: (0, i, 0))] * 3,
        out_shape=[jax.ShapeDtypeStruct((NC, N, CW), jnp.float32)] * 3,
    )(x, wc, bc)


def _sc_body(q_hbm, k_hbm, v_hbm, src_hbm, dst_hbm, z1_hbm, z2_hbm, out_hbm,
             qb0, kb0, vb0, qb1, kb1, vb1, den0, den1,
             db0, db1, sb0, sb1, dbt, sbt, sbig, dbig,
             acc, dacc, gsem0, gsem1, ssem0, ssem1):
    c = lax.axis_index("c")
    s = lax.axis_index("s")
    qp = q_hbm.at[c]
    kp = k_hbm.at[c]
    vp = v_hbm.at[c]
    qb = (qb0, qb1)
    kb = (kb0, kb1)
    vb = (vb0, vb1)
    den = (den0, den1)
    db = (db0, db1)
    sb = (sb0, sb1)
    gsem = (gsem0, gsem1)
    ssem = (ssem0, ssem1)

    pltpu.sync_copy(z1_hbm, acc.at[pl.ds(s * RPT, RPT)])
    pltpu.sync_copy(z2_hbm, dacc.at[pl.ds(s * RPT, RPT)])
    pltpu.sync_copy(z2_hbm.at[pl.ds(0, C)], den0)
    pltpu.sync_copy(z2_hbm.at[pl.ds(0, C)], den1)
    plsc.subcore_barrier()

    iot = lax.iota(jnp.int32, L)
    ebase = s * EPT

    def load_idx(ci, b):
        k = ci & (BF - 1)

        @pl.when(k == 0)
        def _():
            off = pl.multiple_of(ebase + ci * C, 8)
            pltpu.sync_copy(src_hbm.at[pl.ds(off, BF * C)], sbig)
            pltpu.sync_copy(dst_hbm.at[pl.ds(off, BF * C)], dbig)

        for m in range(C // L):
            sl = pl.ds(k * C + m * L, L)
            sb[b][0, pl.ds(m * L, L)] = sbig[sl]
            db[b][0, pl.ds(m * L, L)] = dbig[sl]

    def fire_gathers(b):
        pltpu.async_copy(qp.at[db[b].at[0]], qb[b], gsem[b])
        pltpu.async_copy(kp.at[sb[b].at[0]], kb[b], gsem[b])
        pltpu.async_copy(vp.at[sb[b].at[0]], vb[b], gsem[b])

    def drain_gathers(b):
        pltpu.make_async_copy(qp.at[db[b].at[0]], qb[b], gsem[b]).wait()
        pltpu.make_async_copy(kp.at[sb[b].at[0]], kb[b], gsem[b]).wait()
        pltpu.make_async_copy(vp.at[sb[b].at[0]], vb[b], gsem[b]).wait()

    def fire_scatters(b):
        pltpu.async_copy(vb[b], acc.at[db[b].at[0]], ssem[b], add=True)
        pltpu.async_copy(den[b], dacc.at[db[b].at[0]], ssem[b], add=True)

    def drain_scatters(b):
        pltpu.make_async_copy(vb[b], acc.at[pl.ds(0, C)], ssem[b]).wait()
        pltpu.make_async_copy(den[b], dacc.at[pl.ds(0, C)], ssem[b]).wait()

    def compute(b, ngrp):
        def grp(g, carry2):
            er = g * L + iot

            def score_m(mq, saccs):
                out = list(saccs)
                for u in range(UNR):
                    colp = iot ^ (mq * UNR + u)
                    for h in range(HPC):
                        sacc = out[h]
                        for hi in (0, L):
                            col = colp + (h * DPH + hi)
                            qv = plsc.load_gather(qb[b], [er, col])
                            kv = plsc.load_gather(kb[b], [er, col])
                            sacc = sacc + qv * kv
                        out[h] = sacc
                return tuple(out)

            zero = jnp.zeros((L,), jnp.float32)
            saccs = plsc.parallel_loop(0, L // UNR, carry=(zero,) * HPC)(score_m)
            ws = []
            for h in range(HPC):
                w = jnp.exp(jnp.clip(saccs[h] * _INV_SQRT_D, -5.0, 5.0))
                ws.append(w)
                plsc.store_scatter(den[b], [er, jnp.full((L,), h, jnp.int32)], w)

            def msg_m(mq, carry3):
                for u in range(UNR):
                    colp = iot ^ (mq * UNR + u)
                    for h in range(HPC):
                        for hi in (0, L):
                            col = colp + (h * DPH + hi)
                            vv = plsc.load_gather(vb[b], [er, col])
                            plsc.store_scatter(vb[b], [er, col], vv * ws[h])
                return carry3

            plsc.parallel_loop(0, L // UNR, carry=jnp.int32(0))(msg_m)
            return carry2

        plsc.parallel_loop(0, ngrp, carry=jnp.int32(0))(grp)

    load_idx(0, 0)
    fire_gathers(0)

    def pair(it, carry):
        @pl.when(it > 0)
        def _():
            drain_scatters(1)
        load_idx(2 * it + 1, 1)
        fire_gathers(1)
        drain_gathers(0)
        compute(0, C // L)
        fire_scatters(0)
        @pl.when(it < NPAIR - 1)
        def _():
            drain_scatters(0)
            load_idx(2 * it + 2, 0)
            fire_gathers(0)
        drain_gathers(1)
        compute(1, C // L)
        fire_scatters(1)
        return carry

    lax.fori_loop(0, NPAIR, pair, 0)

    drain_scatters(0)
    toff = pl.multiple_of(ebase + NCHUNK * C, 8)
    pltpu.sync_copy(src_hbm.at[pl.ds(toff, TAILE)], sbt.at[0])
    pltpu.sync_copy(dst_hbm.at[pl.ds(toff, TAILE)], dbt.at[0])
    cp1 = pltpu.async_copy(qp.at[dbt.at[0]], qb0.at[pl.ds(0, TAILE)], gsem0)
    cp2 = pltpu.async_copy(kp.at[sbt.at[0]], kb0.at[pl.ds(0, TAILE)], gsem0)
    cp3 = pltpu.async_copy(vp.at[sbt.at[0]], vb0.at[pl.ds(0, TAILE)], gsem0)
    cp1.wait()
    cp2.wait()
    cp3.wait()
    compute(0, TAILE // L)
    pltpu.async_copy(vb0.at[pl.ds(0, TAILE)], acc.at[dbt.at[0]], ssem0, add=True)
    pltpu.async_copy(den0.at[pl.ds(0, TAILE)], dacc.at[dbt.at[0]], ssem0, add=True)
    pltpu.make_async_copy(vb0.at[pl.ds(0, TAILE)], acc.at[pl.ds(0, TAILE)], ssem0).wait()
    pltpu.make_async_copy(den0.at[pl.ds(0, TAILE)], dacc.at[pl.ds(0, TAILE)], ssem0).wait()
    drain_scatters(1)
    plsc.subcore_barrier()

    def norm_rows(rn, nrows):
        pltpu.sync_copy(acc.at[pl.ds(rn, nrows)], qb0.at[pl.ds(0, nrows)])
        pltpu.sync_copy(dacc.at[pl.ds(rn, nrows)], den0.at[pl.ds(0, nrows)])
        for g2 in range(nrows // L):
            er2 = g2 * L + iot
            recs = [1.0 / (plsc.load_gather(
                den0, [er2, jnp.full((L,), h, jnp.int32)]) + 1e-16)
                    for h in range(HPC)]

            def nm(m, carry2):
                colp = iot ^ m
                for jv in range(CW // L):
                    col = colp + jv * L
                    v = plsc.load_gather(qb0, [er2, col])
                    plsc.store_scatter(qb0, [er2, col], v * recs[jv // 2])
                return carry2

            plsc.parallel_loop(0, L, carry=jnp.int32(0))(nm)
        pltpu.sync_copy(qb0.at[pl.ds(0, nrows)],
                        out_hbm.at[pl.ds(rn, nrows), pl.ds(c * CW, CW)])

    def norm(ch, carry):
        norm_rows(s * NRPT + ch * C, C)
        return carry

    lax.fori_loop(0, NRCH, norm, 0)

    @pl.when(s == NS - 1)
    def _norm_tail():
        norm_rows(NS * NRPT, NTAIL)


_sc_attn = pl.kernel(
    _sc_body,
    out_type=jax.ShapeDtypeStruct((N, QKV), jnp.float32),
    mesh=plsc.VectorSubcoreMesh(core_axis_name="c", subcore_axis_name="s",
                                num_cores=NC, num_subcores=NS),
    scratch_types=[
        pltpu.VMEM((C, CW), jnp.float32),
        pltpu.VMEM((C, CW), jnp.float32),
        pltpu.VMEM((C, CW), jnp.float32),
        pltpu.VMEM((C, CW), jnp.float32),
        pltpu.VMEM((C, CW), jnp.float32),
        pltpu.VMEM((C, CW), jnp.float32),
        pltpu.VMEM((C, 8), jnp.float32),
        pltpu.VMEM((C, 8), jnp.float32),
        pltpu.VMEM((1, C), jnp.int32),
        pltpu.VMEM((1, C), jnp.int32),
        pltpu.VMEM((1, C), jnp.int32),
        pltpu.VMEM((1, C), jnp.int32),
        pltpu.VMEM((1, 16), jnp.int32),
        pltpu.VMEM((1, 16), jnp.int32),
        pltpu.VMEM((BF * C,), jnp.int32),
        pltpu.VMEM((BF * C,), jnp.int32),
        pltpu.VMEM_SHARED((N, CW), jnp.float32),
        pltpu.VMEM_SHARED((N, 8), jnp.float32),
        pltpu.SemaphoreType.DMA,
        pltpu.SemaphoreType.DMA,
        pltpu.SemaphoreType.DMA,
        pltpu.SemaphoreType.DMA,
    ],
    compiler_params=pltpu.CompilerParams(use_tc_tiling_on_sc=False,
                                         needs_layout_passes=False),
)


def kernel(x, edge_index, Wq, bq, Wk, bk, Wv, bv):
    wc = jnp.concatenate([Wq, Wk, Wv], axis=1)
    bc = jnp.concatenate([bq, bk, bv]).reshape(1, 3 * QKV)
    q, k, v = _tc_qkv(x, wc, bc)
    z1 = jnp.zeros((RPT, CW), jnp.float32)
    z2 = jnp.zeros((RPT, 8), jnp.float32)
    return _sc_attn(q, k, v, edge_index[0], edge_index[1], z1, z2)
